# baseline (speedup 1.0000x reference)
import jax
import jax.numpy as jnp
from jax import lax
from jax.experimental import pallas as pl
from jax.experimental.pallas import tpu as pltpu
from jax.experimental.pallas import tpu_sc as plsc

N_NODES = 10000
D_FEAT = 128
N_BLOCKS = 500
N_BLOCKS_PAD = 512
N_GRAPHS = 16

IDXW = 128
GROUPS_PER_SUB = 5
ROWS_PER_SUB = IDXW * GROUPS_PER_SUB
VALID_LAST = N_NODES - 15 * ROWS_PER_SUB

D_HALF = D_FEAT // 2


def _sc_body(h0_hbm, bid_hbm, blk_out,
             rows_v, idxb_v, zero_v, blk_acc, sems, sem_i):
    c = lax.axis_index("c")
    s = lax.axis_index("s")
    col0 = c * D_HALF
    r0 = s * ROWS_PER_SUB

    @pl.when(s < 15)
    def _():
        for j in range(GROUPS_PER_SUB):
            pltpu.async_copy(bid_hbm.at[pl.ds(r0 + IDXW * j, IDXW)],
                             idxb_v.at[j], sem_i)
        for j in range(GROUPS_PER_SUB):
            pltpu.async_copy(
                h0_hbm.at[pl.ds(r0 + IDXW * j, IDXW), pl.ds(col0, D_HALF)],
                rows_v.at[pl.ds(IDXW * j, IDXW)], sems.at[j])

    @pl.when(s == 15)
    def _():
        nt = VALID_LAST - IDXW * 3
        for j in range(3):
            pltpu.async_copy(bid_hbm.at[pl.ds(r0 + IDXW * j, IDXW)],
                             idxb_v.at[j], sem_i)
        pltpu.async_copy(bid_hbm.at[pl.ds(r0 + IDXW * 3, nt)],
                         idxb_v.at[3, pl.ds(0, nt)], sem_i)
        for j in range(3):
            pltpu.async_copy(
                h0_hbm.at[pl.ds(r0 + IDXW * j, IDXW), pl.ds(col0, D_HALF)],
                rows_v.at[pl.ds(IDXW * j, IDXW)], sems.at[j])
        pltpu.async_copy(h0_hbm.at[pl.ds(r0 + IDXW * 3, nt),
                                   pl.ds(col0, D_HALF)],
                         rows_v.at[pl.ds(IDXW * 3, nt)], sems.at[3])

    z16 = jnp.zeros((16,), jnp.float32)
    for i in range(32):
        for j in range(4):
            zero_v[i, pl.ds(16 * j, 16)] = z16

    @pl.when(s == 15)
    def _():
        dump = jnp.full((16,), N_BLOCKS_PAD - 1, jnp.int32)
        for k in range(1, 8):
            idxb_v[3, pl.ds(16 * k, 16)] = dump
        for k in range(8):
            idxb_v[4, pl.ds(16 * k, 16)] = dump

    pltpu.sync_copy(zero_v, blk_acc.at[pl.ds(s * 32, 32)])

    plsc.subcore_barrier()

    @pl.when(s < 15)
    def _():
        for j in range(GROUPS_PER_SUB):
            pltpu.make_async_copy(bid_hbm.at[pl.ds(r0 + IDXW * j, IDXW)],
                                  idxb_v.at[j], sem_i).wait()

    @pl.when(s == 15)
    def _():
        for j in range(3):
            pltpu.make_async_copy(bid_hbm.at[pl.ds(r0 + IDXW * j, IDXW)],
                                  idxb_v.at[j], sem_i).wait()
        nt = VALID_LAST - IDXW * 3
        pltpu.make_async_copy(bid_hbm.at[pl.ds(r0 + IDXW * 3, nt)],
                              idxb_v.at[3, pl.ds(0, nt)], sem_i).wait()

    descs = []
    for j in range(GROUPS_PER_SUB):
        rows = rows_v.at[pl.ds(IDXW * j, IDXW)]

        @pl.when(s < 15)
        def _(j=j, rows=rows):
            pltpu.make_async_copy(
                h0_hbm.at[pl.ds(r0 + IDXW * j, IDXW), pl.ds(col0, D_HALF)],
                rows, sems.at[j]).wait()

        if j <= 2:
            @pl.when(s == 15)
            def _(j=j, rows=rows):
                pltpu.make_async_copy(
                    h0_hbm.at[pl.ds(r0 + IDXW * j, IDXW),
                              pl.ds(col0, D_HALF)],
                    rows, sems.at[j]).wait()
        elif j == 3:
            @pl.when(s == 15)
            def _(j=j):
                nt = VALID_LAST - IDXW * 3
                pltpu.make_async_copy(
                    h0_hbm.at[pl.ds(r0 + IDXW * 3, nt), pl.ds(col0, D_HALF)],
                    rows_v.at[pl.ds(IDXW * 3, nt)], sems.at[3]).wait()

        descs.append(pltpu.async_copy(rows, blk_acc.at[idxb_v.at[j]],
                                      sem_i, add=True))
    for d in descs:
        d.wait()

    plsc.subcore_barrier()

    @pl.when(s < 15)
    def _():
        pltpu.sync_copy(blk_acc.at[pl.ds(s * 32, 32)],
                        blk_out.at[pl.ds(s * 32, 32), pl.ds(col0, D_HALF)])

    @pl.when(s == 15)
    def _():
        pltpu.sync_copy(blk_acc.at[pl.ds(480, N_BLOCKS - 480)],
                        blk_out.at[pl.ds(480, N_BLOCKS - 480),
                                   pl.ds(col0, D_HALF)])


def _tc_body(gid_ref, x_ref, out_ref, x_copy_ref):
    ids = gid_ref[...].reshape(N_NODES, 1)
    onehot = (ids == lax.broadcasted_iota(jnp.int32, (1, N_GRAPHS), 1)
              ).astype(jnp.float32)
    out_ref[...] = lax.dot_general(onehot, x_ref[...],
                                   (((0,), (0,)), ((), ())),
                                   preferred_element_type=jnp.float32)
    x_copy_ref[...] = x_ref[...]


@jax.jit
def _encoder_pool(h0, bid, gid, z):
    mesh = plsc.VectorSubcoreMesh(core_axis_name="c", subcore_axis_name="s")
    sc = pl.kernel(
        _sc_body,
        out_type=jax.ShapeDtypeStruct((N_BLOCKS, D_FEAT), jnp.float32),
        mesh=mesh,
        compiler_params=pltpu.CompilerParams(use_tc_tiling_on_sc=False),
        scratch_types=[
            pltpu.VMEM((ROWS_PER_SUB, D_HALF), jnp.float32),
            pltpu.VMEM((GROUPS_PER_SUB, IDXW), jnp.int32),
            pltpu.VMEM((32, D_HALF), jnp.float32),
            pltpu.VMEM_SHARED((N_BLOCKS_PAD, D_HALF), jnp.float32),
            pltpu.SemaphoreType.DMA((GROUPS_PER_SUB,)),
            pltpu.SemaphoreType.DMA,
        ],
    )
    block_repr = sc(h0, bid)

    graph_repr, h0_out = pl.pallas_call(
        _tc_body,
        out_shape=(
            jax.ShapeDtypeStruct((N_GRAPHS, D_FEAT), jnp.float32),
            jax.ShapeDtypeStruct((N_NODES, D_FEAT), jnp.float32),
        ),
    )(gid, h0)

    return block_repr, graph_repr, h0_out, z


def kernel(H_0, Z, block_id, batch_id, edges, edge_attr):
    block_repr, graph_repr, unit_repr, pred_Z = _encoder_pool(
        H_0, block_id.astype(jnp.int32), batch_id.astype(jnp.int32), Z)
    return (unit_repr, block_repr, graph_repr, pred_Z)


# speedup vs baseline: 1.0062x; 1.0062x over previous
import jax
import jax.numpy as jnp
from jax import lax
from jax.experimental import pallas as pl
from jax.experimental.pallas import tpu as pltpu
from jax.experimental.pallas import tpu_sc as plsc

N_NODES = 10000
D_FEAT = 128
N_BLOCKS = 500
N_BLOCKS_PAD = 512
N_GRAPHS = 16

IDXW = 128
GROUPS_PER_SUB = 5
ROWS_PER_SUB = IDXW * GROUPS_PER_SUB
VALID_LAST = N_NODES - 15 * ROWS_PER_SUB

D_HALF = D_FEAT // 2


def _sc_body(h0_hbm, bid_hbm, blk_out,
             rows_v, idxb_v, zero_v, blk_acc, sems, sem_i):
    c = lax.axis_index("c")
    s = lax.axis_index("s")
    col0 = c * D_HALF
    r0 = s * ROWS_PER_SUB

    @pl.when(s < 15)
    def _():
        for j in range(GROUPS_PER_SUB):
            pltpu.async_copy(bid_hbm.at[pl.ds(r0 + IDXW * j, IDXW)],
                             idxb_v.at[j], sem_i)
        for j in range(GROUPS_PER_SUB):
            pltpu.async_copy(
                h0_hbm.at[pl.ds(r0 + IDXW * j, IDXW), pl.ds(col0, D_HALF)],
                rows_v.at[pl.ds(IDXW * j, IDXW)], sems.at[j])

    @pl.when(s == 15)
    def _():
        nt = VALID_LAST - IDXW * 3
        for j in range(3):
            pltpu.async_copy(bid_hbm.at[pl.ds(r0 + IDXW * j, IDXW)],
                             idxb_v.at[j], sem_i)
        pltpu.async_copy(bid_hbm.at[pl.ds(r0 + IDXW * 3, nt)],
                         idxb_v.at[3, pl.ds(0, nt)], sem_i)
        for j in range(3):
            pltpu.async_copy(
                h0_hbm.at[pl.ds(r0 + IDXW * j, IDXW), pl.ds(col0, D_HALF)],
                rows_v.at[pl.ds(IDXW * j, IDXW)], sems.at[j])
        pltpu.async_copy(h0_hbm.at[pl.ds(r0 + IDXW * 3, nt),
                                   pl.ds(col0, D_HALF)],
                         rows_v.at[pl.ds(IDXW * 3, nt)], sems.at[3])

    z16 = jnp.zeros((16,), jnp.float32)
    for i in range(32):
        for j in range(4):
            zero_v[i, pl.ds(16 * j, 16)] = z16

    @pl.when(s == 15)
    def _():
        dump = jnp.full((16,), N_BLOCKS_PAD - 1, jnp.int32)
        for k in range(1, 8):
            idxb_v[3, pl.ds(16 * k, 16)] = dump
        for k in range(8):
            idxb_v[4, pl.ds(16 * k, 16)] = dump

    pltpu.sync_copy(zero_v, blk_acc.at[pl.ds(s * 32, 32)])

    plsc.subcore_barrier()

    @pl.when(s < 15)
    def _():
        for j in range(GROUPS_PER_SUB):
            pltpu.make_async_copy(bid_hbm.at[pl.ds(r0 + IDXW * j, IDXW)],
                                  idxb_v.at[j], sem_i).wait()

    @pl.when(s == 15)
    def _():
        for j in range(3):
            pltpu.make_async_copy(bid_hbm.at[pl.ds(r0 + IDXW * j, IDXW)],
                                  idxb_v.at[j], sem_i).wait()
        nt = VALID_LAST - IDXW * 3
        pltpu.make_async_copy(bid_hbm.at[pl.ds(r0 + IDXW * 3, nt)],
                              idxb_v.at[3, pl.ds(0, nt)], sem_i).wait()

    descs = []
    for j in range(GROUPS_PER_SUB):
        rows = rows_v.at[pl.ds(IDXW * j, IDXW)]

        @pl.when(jnp.logical_or(s < 15, j <= 2))
        def _(j=j, rows=rows):
            pltpu.make_async_copy(
                h0_hbm.at[pl.ds(r0 + IDXW * j, IDXW), pl.ds(col0, D_HALF)],
                rows, sems.at[j]).wait()

        if j == 3:
            @pl.when(s == 15)
            def _(j=j):
                nt = VALID_LAST - IDXW * 3
                pltpu.make_async_copy(
                    h0_hbm.at[pl.ds(r0 + IDXW * 3, nt), pl.ds(col0, D_HALF)],
                    rows_v.at[pl.ds(IDXW * 3, nt)], sems.at[3]).wait()

        descs.append(pltpu.async_copy(rows, blk_acc.at[idxb_v.at[j]],
                                      sem_i, add=True))
    for d in descs:
        d.wait()

    plsc.subcore_barrier()

    @pl.when(s < 15)
    def _():
        pltpu.sync_copy(blk_acc.at[pl.ds(s * 32, 32)],
                        blk_out.at[pl.ds(s * 32, 32), pl.ds(col0, D_HALF)])

    @pl.when(s == 15)
    def _():
        pltpu.sync_copy(blk_acc.at[pl.ds(480, N_BLOCKS - 480)],
                        blk_out.at[pl.ds(480, N_BLOCKS - 480),
                                   pl.ds(col0, D_HALF)])


def _tc_body(gid_ref, x_ref, out_ref, x_copy_ref):
    ids = gid_ref[...].reshape(N_NODES, 1)
    onehot = (ids == lax.broadcasted_iota(jnp.int32, (1, N_GRAPHS), 1)
              ).astype(jnp.float32)
    out_ref[...] = lax.dot_general(onehot, x_ref[...],
                                   (((0,), (0,)), ((), ())),
                                   preferred_element_type=jnp.float32)
    x_copy_ref[...] = x_ref[...]


@jax.jit
def _encoder_pool(h0, bid, gid, z):
    mesh = plsc.VectorSubcoreMesh(core_axis_name="c", subcore_axis_name="s")
    sc = pl.kernel(
        _sc_body,
        out_type=jax.ShapeDtypeStruct((N_BLOCKS, D_FEAT), jnp.float32),
        mesh=mesh,
        compiler_params=pltpu.CompilerParams(use_tc_tiling_on_sc=False),
        scratch_types=[
            pltpu.VMEM((ROWS_PER_SUB, D_HALF), jnp.float32),
            pltpu.VMEM((GROUPS_PER_SUB, IDXW), jnp.int32),
            pltpu.VMEM((32, D_HALF), jnp.float32),
            pltpu.VMEM_SHARED((N_BLOCKS_PAD, D_HALF), jnp.float32),
            pltpu.SemaphoreType.DMA((GROUPS_PER_SUB,)),
            pltpu.SemaphoreType.DMA,
        ],
    )
    block_repr = sc(h0, bid)

    graph_repr, h0_out = pl.pallas_call(
        _tc_body,
        out_shape=(
            jax.ShapeDtypeStruct((N_GRAPHS, D_FEAT), jnp.float32),
            jax.ShapeDtypeStruct((N_NODES, D_FEAT), jnp.float32),
        ),
    )(gid, h0)

    return block_repr, graph_repr, h0_out, z


def kernel(H_0, Z, block_id, batch_id, edges, edge_attr):
    block_repr, graph_repr, unit_repr, pred_Z = _encoder_pool(
        H_0, block_id.astype(jnp.int32), batch_id.astype(jnp.int32), Z)
    return (unit_repr, block_repr, graph_repr, pred_Z)


# speedup vs baseline: 1.0078x; 1.0015x over previous
import jax
import jax.numpy as jnp
from jax import lax
from jax.experimental import pallas as pl
from jax.experimental.pallas import tpu as pltpu
from jax.experimental.pallas import tpu_sc as plsc

N_NODES = 10000
D_FEAT = 128
N_BLOCKS = 500
N_BLOCKS_PAD = 512
N_GRAPHS = 16

IDXW = 128
GROUPS_PER_SUB = 5
ROWS_PER_SUB = IDXW * GROUPS_PER_SUB
VALID_LAST = N_NODES - 15 * ROWS_PER_SUB

D_HALF = D_FEAT // 2


def _sc_body(h0_hbm, bid_hbm, blk_out,
             rows_v, idxb_v, zero_v, blk_acc, sems, sem_i):
    c = lax.axis_index("c")
    s = lax.axis_index("s")
    col0 = c * D_HALF
    r0 = s * ROWS_PER_SUB

    @pl.when(s < 15)
    def _():
        for j in range(GROUPS_PER_SUB):
            pltpu.async_copy(bid_hbm.at[pl.ds(r0 + IDXW * j, IDXW)],
                             idxb_v.at[j], sem_i)
        for j in range(GROUPS_PER_SUB):
            pltpu.async_copy(
                h0_hbm.at[pl.ds(r0 + IDXW * j, IDXW), pl.ds(col0, D_HALF)],
                rows_v.at[pl.ds(IDXW * j, IDXW)], sems.at[j])

    @pl.when(s == 15)
    def _():
        nt = VALID_LAST - IDXW * 3
        for j in range(3):
            pltpu.async_copy(bid_hbm.at[pl.ds(r0 + IDXW * j, IDXW)],
                             idxb_v.at[j], sem_i)
        pltpu.async_copy(bid_hbm.at[pl.ds(r0 + IDXW * 3, nt)],
                         idxb_v.at[3, pl.ds(0, nt)], sem_i)
        for j in range(3):
            pltpu.async_copy(
                h0_hbm.at[pl.ds(r0 + IDXW * j, IDXW), pl.ds(col0, D_HALF)],
                rows_v.at[pl.ds(IDXW * j, IDXW)], sems.at[j])
        pltpu.async_copy(h0_hbm.at[pl.ds(r0 + IDXW * 3, nt),
                                   pl.ds(col0, D_HALF)],
                         rows_v.at[pl.ds(IDXW * 3, nt)], sems.at[3])

    z16 = jnp.zeros((16,), jnp.float32)
    for i in range(32):
        for j in range(4):
            zero_v[i, pl.ds(16 * j, 16)] = z16

    @pl.when(s == 15)
    def _():
        dump = jnp.full((16,), N_BLOCKS_PAD - 1, jnp.int32)
        for k in range(1, 8):
            idxb_v[3, pl.ds(16 * k, 16)] = dump
        for k in range(8):
            idxb_v[4, pl.ds(16 * k, 16)] = dump

    pltpu.sync_copy(zero_v, blk_acc.at[pl.ds(s * 32, 32)])

    plsc.subcore_barrier()

    @pl.when(s < 15)
    def _():
        for j in range(GROUPS_PER_SUB):
            pltpu.make_async_copy(bid_hbm.at[pl.ds(r0 + IDXW * j, IDXW)],
                                  idxb_v.at[j], sem_i).wait()

    @pl.when(s == 15)
    def _():
        for j in range(3):
            pltpu.make_async_copy(bid_hbm.at[pl.ds(r0 + IDXW * j, IDXW)],
                                  idxb_v.at[j], sem_i).wait()
        nt = VALID_LAST - IDXW * 3
        pltpu.make_async_copy(bid_hbm.at[pl.ds(r0 + IDXW * 3, nt)],
                              idxb_v.at[3, pl.ds(0, nt)], sem_i).wait()

    descs = []
    for j in range(GROUPS_PER_SUB):
        rows = rows_v.at[pl.ds(IDXW * j, IDXW)]

        @pl.when(jnp.logical_or(s < 15, j <= 2))
        def _(j=j, rows=rows):
            pltpu.make_async_copy(
                h0_hbm.at[pl.ds(r0 + IDXW * j, IDXW), pl.ds(col0, D_HALF)],
                rows, sems.at[j]).wait()

        if j == 3:
            @pl.when(s == 15)
            def _(j=j):
                nt = VALID_LAST - IDXW * 3
                pltpu.make_async_copy(
                    h0_hbm.at[pl.ds(r0 + IDXW * 3, nt), pl.ds(col0, D_HALF)],
                    rows_v.at[pl.ds(IDXW * 3, nt)], sems.at[3]).wait()

        descs.append(pltpu.async_copy(rows, blk_acc.at[idxb_v.at[j]],
                                      sem_i, add=True))
    for d in descs:
        d.wait()

    plsc.subcore_barrier()

    @pl.when(s < 15)
    def _():
        pltpu.sync_copy(blk_acc.at[pl.ds(s * 32, 32)],
                        blk_out.at[pl.ds(s * 32, 32), pl.ds(col0, D_HALF)])

    @pl.when(s == 15)
    def _():
        pltpu.sync_copy(blk_acc.at[pl.ds(480, N_BLOCKS - 480)],
                        blk_out.at[pl.ds(480, N_BLOCKS - 480),
                                   pl.ds(col0, D_HALF)])


def _tc_body(gid_ref, x_ref, out_ref, x_copy_ref):
    ids = gid_ref[...].reshape(N_NODES, 1)
    onehot = (ids == lax.broadcasted_iota(jnp.int32, (1, N_GRAPHS), 1)
              ).astype(jnp.float32)
    out_ref[...] = lax.dot_general(onehot, x_ref[...],
                                   (((0,), (0,)), ((), ())),
                                   precision=lax.Precision.HIGHEST,
                                   preferred_element_type=jnp.float32)
    x_copy_ref[...] = x_ref[...]


@jax.jit
def _encoder_pool(h0, bid, gid, z):
    mesh = plsc.VectorSubcoreMesh(core_axis_name="c", subcore_axis_name="s")
    sc = pl.kernel(
        _sc_body,
        out_type=jax.ShapeDtypeStruct((N_BLOCKS, D_FEAT), jnp.float32),
        mesh=mesh,
        compiler_params=pltpu.CompilerParams(use_tc_tiling_on_sc=False),
        scratch_types=[
            pltpu.VMEM((ROWS_PER_SUB, D_HALF), jnp.float32),
            pltpu.VMEM((GROUPS_PER_SUB, IDXW), jnp.int32),
            pltpu.VMEM((32, D_HALF), jnp.float32),
            pltpu.VMEM_SHARED((N_BLOCKS_PAD, D_HALF), jnp.float32),
            pltpu.SemaphoreType.DMA((GROUPS_PER_SUB,)),
            pltpu.SemaphoreType.DMA,
        ],
    )
    block_repr = sc(h0, bid)

    graph_repr, h0_out = pl.pallas_call(
        _tc_body,
        out_shape=(
            jax.ShapeDtypeStruct((N_GRAPHS, D_FEAT), jnp.float32),
            jax.ShapeDtypeStruct((N_NODES, D_FEAT), jnp.float32),
        ),
    )(gid, h0)

    return block_repr, graph_repr, h0_out, z


def kernel(H_0, Z, block_id, batch_id, edges, edge_attr):
    block_repr, graph_repr, unit_repr, pred_Z = _encoder_pool(
        H_0, block_id.astype(jnp.int32), batch_id.astype(jnp.int32), Z)
    return (unit_repr, block_repr, graph_repr, pred_Z)
